# baseline (speedup 1.0000x reference)
import functools

import jax
import jax.numpy as jnp
from jax import lax
from jax.experimental import pallas as pl
from jax.experimental.pallas import tpu as pltpu
from jax.experimental.pallas import tpu_sc as plsc

B, L, H = 1024, 50, 768
NC, NS, LANES = 2, 16, 16
NW = NC * NS
TOK_PER_W = (B * L) // NW
C = 32
NCHUNK = TOK_PER_W // C


def _build_kernel():
    mesh = plsc.VectorSubcoreMesh(core_axis_name="c", subcore_axis_name="s")

    @functools.partial(
        pl.kernel,
        mesh=mesh,
        out_type=jax.ShapeDtypeStruct((B * L, H), jnp.float32),
        scratch_types=[
            pltpu.VMEM((TOK_PER_W,), jnp.int32),
            pltpu.VMEM((TOK_PER_W,), jnp.int32),
            pltpu.VMEM((C,), jnp.int32),
            pltpu.VMEM((C, H), jnp.float32),
            pltpu.VMEM((C, H), jnp.float32),
            pltpu.VMEM((C, H), jnp.float32),
            pltpu.SemaphoreType.DMA,
            pltpu.SemaphoreType.DMA,
            pltpu.SemaphoreType.DMA,
        ],
    )
    def gather_sum(x_hbm, xseg_hbm, tok_hbm, pos_hbm, seg_hbm, out_hbm,
                   x_v, xseg_v, pidx_v, tok_b, pos_b, seg_b,
                   sem_t, sem_p, sem_s):
        wid = lax.axis_index("s") * NC + lax.axis_index("c")
        base = wid * TOK_PER_W
        pltpu.sync_copy(x_hbm.at[pl.ds(base, TOK_PER_W)], x_v)
        pltpu.sync_copy(xseg_hbm.at[pl.ds(base, TOK_PER_W)], xseg_v)

        def chunk_body(ci, carry):
            t0 = pl.multiple_of(ci * C, C)
            for k in range(C // LANES):
                tv = t0 + k * LANES + lax.iota(jnp.int32, LANES)
                lv = lax.rem(tv, L)
                xv = x_v[pl.ds(t0 + k * LANES, LANES)]
                pv = jnp.where(xv == 0, 0, lv + 1).astype(jnp.int32)
                pidx_v[pl.ds(k * LANES, LANES)] = pv

            cp_t = pltpu.async_copy(tok_hbm.at[x_v.at[pl.ds(t0, C)]],
                                    tok_b, sem_t)
            cp_p = pltpu.async_copy(pos_hbm.at[pidx_v], pos_b, sem_p)
            cp_s = pltpu.async_copy(seg_hbm.at[xseg_v.at[pl.ds(t0, C)]],
                                    seg_b, sem_s)
            cp_t.wait()
            cp_p.wait()
            cp_s.wait()

            def row_body(tj, c2):
                for d in range(H // LANES):
                    sl = pl.ds(d * LANES, LANES)
                    tok_b[tj, sl] = tok_b[tj, sl] + pos_b[tj, sl] + seg_b[tj, sl]
                return c2
            lax.fori_loop(0, C, row_body, 0)

            pltpu.sync_copy(tok_b, out_hbm.at[pl.ds(base + t0, C)])
            return carry

        lax.fori_loop(0, NCHUNK, chunk_body, 0)

    return gather_sum


_gather_sum = _build_kernel()


@jax.jit
def kernel(x, x_seg, token_table, pos_table, seg_table):
    out = _gather_sum(x.reshape(-1), x_seg.reshape(-1),
                      token_table, pos_table, seg_table)
    return out.reshape(B, L, H)


# speedup vs baseline: 1.7243x; 1.7243x over previous
import functools

import jax
import jax.numpy as jnp
from jax import lax
from jax.experimental import pallas as pl
from jax.experimental.pallas import tpu as pltpu
from jax.experimental.pallas import tpu_sc as plsc

B, L, H = 1024, 50, 768
NC, NS, LANES = 2, 16, 16
NW = NC * NS
TOK_PER_W = (B * L) // NW
C = 16
NCHUNK = TOK_PER_W // C
POS_ROWS = 56
UNROLL = 4


def _build_kernel():
    mesh = plsc.VectorSubcoreMesh(core_axis_name="c", subcore_axis_name="s")

    @functools.partial(
        pl.kernel,
        mesh=mesh,
        out_type=jax.ShapeDtypeStruct((B * L, H), jnp.float32),
        scratch_types=[
            pltpu.VMEM((TOK_PER_W,), jnp.int32),
            pltpu.VMEM((TOK_PER_W,), jnp.int32),
            pltpu.VMEM((POS_ROWS, H), jnp.float32),
            pltpu.VMEM((16,), jnp.int32),
            pltpu.VMEM((16, H), jnp.float32),
            pltpu.VMEM((2, C, H), jnp.float32),
            pltpu.SemaphoreType.DMA,
            pltpu.SemaphoreType.DMA,
            pltpu.SemaphoreType.DMA,
            pltpu.SemaphoreType.DMA,
        ],
    )
    def gather_sum(x_hbm, xseg_hbm, tok_hbm, pos_hbm, seg_hbm, out_hbm,
                   x_v, xseg_v, pos_v, sidx_v, seg_v, acc2,
                   sem_g0, sem_g1, sem_w0, sem_w1):
        sem_g = (sem_g0, sem_g1)
        sem_w = (sem_w0, sem_w1)

        wid = lax.axis_index("s") * NC + lax.axis_index("c")
        base = wid * TOK_PER_W
        pltpu.sync_copy(x_hbm.at[pl.ds(base, TOK_PER_W)], x_v)
        pltpu.sync_copy(xseg_hbm.at[pl.ds(base, TOK_PER_W)], xseg_v)
        pltpu.sync_copy(pos_hbm.at[pl.ds(0, POS_ROWS)], pos_v)
        iv = lax.iota(jnp.int32, 16)
        sidx_v[pl.ds(0, 16)] = jnp.where(iv < 3, iv, 0)
        pltpu.async_copy(seg_hbm.at[sidx_v], seg_v, sem_g0).wait()

        def fire_tok(ci, slot):
            t0 = pl.multiple_of(ci * C, C)
            pltpu.async_copy(tok_hbm.at[x_v.at[pl.ds(t0, C)]],
                             acc2.at[slot], sem_g[slot])

        def process(ci, slot, fire_next, wait_wb):
            nslot = 1 - slot
            if fire_next:
                if wait_wb:
                    pltpu.make_async_copy(
                        acc2.at[nslot], out_hbm.at[pl.ds(0, C)],
                        sem_w[nslot]).wait()
                fire_tok(ci + 1, nslot)
            t0 = pl.multiple_of(ci * C, C)
            tv = t0 + lax.iota(jnp.int32, LANES)
            lv = lax.rem(tv, L)
            xv = x_v[pl.ds(t0, LANES)]
            sv = xseg_v[pl.ds(t0, LANES)]
            pt = jnp.where(xv == 0, 0, lv + 1).astype(jnp.int32)
            prow = [pt[j] for j in range(C)]
            srow = [sv[j] for j in range(C)]
            pltpu.make_async_copy(tok_hbm.at[pl.ds(0, C)],
                                  acc2.at[slot], sem_g[slot]).wait()

            def col_body(dd, carry):
                for u in range(UNROLL):
                    d = dd * UNROLL + u
                    sl = pl.ds(pl.multiple_of(d * LANES, LANES), LANES)
                    for j in range(C):
                        plsc.addupdate(acc2.at[slot, j, sl],
                                       pos_v[prow[j], sl] + seg_v[srow[j], sl])
                return carry
            lax.fori_loop(0, H // (LANES * UNROLL), col_body, 0)

            pltpu.async_copy(acc2.at[slot], out_hbm.at[pl.ds(base + t0, C)],
                             sem_w[slot])

        fire_tok(0, 0)
        process(0, 0, fire_next=True, wait_wb=False)

        def pair_body(p, carry):
            ci = 1 + 2 * p
            process(ci, 1, fire_next=True, wait_wb=True)
            process(ci + 1, 0, fire_next=True, wait_wb=True)
            return carry
        lax.fori_loop(0, (NCHUNK - 2) // 2, pair_body, 0)

        process(NCHUNK - 1, 1, fire_next=False, wait_wb=False)

        pltpu.make_async_copy(acc2.at[0], out_hbm.at[pl.ds(0, C)],
                              sem_w[0]).wait()
        pltpu.make_async_copy(acc2.at[1], out_hbm.at[pl.ds(0, C)],
                              sem_w[1]).wait()

    return gather_sum


_gather_sum = _build_kernel()


@jax.jit
def kernel(x, x_seg, token_table, pos_table, seg_table):
    out = _gather_sum(x.reshape(-1), x_seg.reshape(-1),
                      token_table, pos_table, seg_table)
    return out.reshape(B, L, H)


# speedup vs baseline: 1.8501x; 1.0729x over previous
import functools

import jax
import jax.numpy as jnp
from jax import lax
from jax.experimental import pallas as pl
from jax.experimental.pallas import tpu as pltpu
from jax.experimental.pallas import tpu_sc as plsc

B, L, H = 1024, 50, 768
NC, NS, LANES = 2, 16, 16
NW = NC * NS
TOK_PER_W = (B * L) // NW
C = 16
NCHUNK = TOK_PER_W // C
POS_ROWS = 56


def _build_kernel():
    mesh = plsc.VectorSubcoreMesh(core_axis_name="c", subcore_axis_name="s")

    @functools.partial(
        pl.kernel,
        mesh=mesh,
        out_type=jax.ShapeDtypeStruct((B * L, H), jnp.float32),
        scratch_types=[
            pltpu.VMEM((TOK_PER_W,), jnp.int32),
            pltpu.VMEM((TOK_PER_W,), jnp.int32),
            pltpu.VMEM((POS_ROWS, H), jnp.float32),
            pltpu.VMEM((16,), jnp.int32),
            pltpu.VMEM((16, H), jnp.float32),
            pltpu.VMEM((2 * C, H), jnp.float32),
            pltpu.SMEM((2 * C,), jnp.int32),
            pltpu.SemaphoreType.DMA,
            pltpu.SemaphoreType.DMA,
            pltpu.SemaphoreType.DMA,
            pltpu.SemaphoreType.DMA,
        ],
    )
    def gather_sum(x_hbm, xseg_hbm, tok_hbm, pos_hbm, seg_hbm, out_hbm,
                   x_v, xseg_v, pos_v, sidx_v, seg_v, acc2, rows_m,
                   sem_g0, sem_g1, sem_w0, sem_w1):
        sem_g = (sem_g0, sem_g1)
        sem_w = (sem_w0, sem_w1)

        wid = lax.axis_index("s") * NC + lax.axis_index("c")
        base = wid * TOK_PER_W
        pltpu.sync_copy(x_hbm.at[pl.ds(base, TOK_PER_W)], x_v)
        pltpu.sync_copy(xseg_hbm.at[pl.ds(base, TOK_PER_W)], xseg_v)
        pltpu.sync_copy(pos_hbm.at[pl.ds(0, POS_ROWS)], pos_v)
        iv = lax.iota(jnp.int32, 16)
        sidx_v[pl.ds(0, 16)] = jnp.where(iv < 3, iv, 0)
        pltpu.async_copy(seg_hbm.at[sidx_v], seg_v, sem_g0).wait()

        def fire_tok(ci, slot):
            t0 = pl.multiple_of(ci * C, C)
            pltpu.async_copy(tok_hbm.at[x_v.at[pl.ds(t0, C)]],
                             acc2.at[pl.ds(slot * C, C)], sem_g[slot])

        def process(ci, slot, fire_next, wait_wb):
            nslot = 1 - slot
            if fire_next:
                if wait_wb:
                    pltpu.make_async_copy(
                        acc2.at[pl.ds(nslot * C, C)], out_hbm.at[pl.ds(0, C)],
                        sem_w[nslot]).wait()
                fire_tok(ci + 1, nslot)
            t0 = pl.multiple_of(ci * C, C)
            tv = t0 + lax.iota(jnp.int32, LANES)
            lv = lax.rem(tv, L)
            xv = x_v[pl.ds(t0, LANES)]
            sv = xseg_v[pl.ds(t0, LANES)]
            pt = jnp.where(xv == 0, 0, lv + 1).astype(jnp.int32)
            for j in range(C):
                rows_m[j] = pt[j]
                rows_m[C + j] = sv[j]
            pltpu.make_async_copy(tok_hbm.at[pl.ds(0, C)],
                                  acc2.at[pl.ds(slot * C, C)],
                                  sem_g[slot]).wait()

            def row_body(tj, carry):
                prow = rows_m[tj]
                srow = rows_m[C + tj]
                for d in range(H // LANES):
                    sl = pl.ds(d * LANES, LANES)
                    plsc.addupdate(acc2.at[slot * C + tj, sl],
                                   pos_v[prow, sl] + seg_v[srow, sl])
                return carry
            lax.fori_loop(0, C, row_body, 0)

            pltpu.async_copy(acc2.at[pl.ds(slot * C, C)],
                             out_hbm.at[pl.ds(base + t0, C)], sem_w[slot])

        fire_tok(0, 0)
        process(0, 0, fire_next=True, wait_wb=False)

        def pair_body(p, carry):
            ci = 1 + 2 * p
            process(ci, 1, fire_next=True, wait_wb=True)
            process(ci + 1, 0, fire_next=True, wait_wb=True)
            return carry
        lax.fori_loop(0, (NCHUNK - 2) // 2, pair_body, 0)

        process(NCHUNK - 1, 1, fire_next=False, wait_wb=False)

        pltpu.make_async_copy(acc2.at[pl.ds(0, C)], out_hbm.at[pl.ds(0, C)],
                              sem_w[0]).wait()
        pltpu.make_async_copy(acc2.at[pl.ds(C, C)], out_hbm.at[pl.ds(0, C)],
                              sem_w[1]).wait()

    return gather_sum


_gather_sum = _build_kernel()


@jax.jit
def kernel(x, x_seg, token_table, pos_table, seg_table):
    out = _gather_sum(x.reshape(-1), x_seg.reshape(-1),
                      token_table, pos_table, seg_table)
    return out.reshape(B, L, H)


# speedup vs baseline: 2.8119x; 1.5199x over previous
import functools

import jax
import jax.numpy as jnp
from jax import lax
from jax.experimental import pallas as pl
from jax.experimental.pallas import tpu as pltpu
from jax.experimental.pallas import tpu_sc as plsc

B, L, H = 1024, 50, 768
NC, NS, LANES = 2, 16, 16
NW = NC * NS
TOK_PER_W = (B * L) // NW
C = 16
NCHUNK = TOK_PER_W // C
POS_ROWS = 56


def _build_kernel():
    mesh = plsc.VectorSubcoreMesh(core_axis_name="c", subcore_axis_name="s")

    @functools.partial(
        pl.kernel,
        mesh=mesh,
        out_type=jax.ShapeDtypeStruct((B * L, H), jnp.float32),
        scratch_types=[
            pltpu.VMEM((TOK_PER_W,), jnp.int32),
            pltpu.VMEM((TOK_PER_W,), jnp.int32),
            pltpu.VMEM((POS_ROWS, H), jnp.float32),
            pltpu.VMEM((16,), jnp.int32),
            pltpu.VMEM((16, H), jnp.float32),
            pltpu.VMEM((2 * C, H), jnp.float32),
            pltpu.SMEM((2 * C,), jnp.int32),
            pltpu.SemaphoreType.DMA,
            pltpu.SemaphoreType.DMA,
            pltpu.SemaphoreType.DMA,
            pltpu.SemaphoreType.DMA,
        ],
    )
    def gather_sum(x_hbm, xseg_hbm, tok_hbm, pos_hbm, seg_hbm, out_hbm,
                   x_v, xseg_v, pos_v, sidx_v, seg_v, acc2, rows_m,
                   sem_g0, sem_g1, sem_w0, sem_w1):
        sem_g = (sem_g0, sem_g1)
        sem_w = (sem_w0, sem_w1)

        wid = lax.axis_index("s") * NC + lax.axis_index("c")
        base = wid * TOK_PER_W
        pltpu.sync_copy(x_hbm.at[pl.ds(base, TOK_PER_W)], x_v)
        pltpu.sync_copy(xseg_hbm.at[pl.ds(base, TOK_PER_W)], xseg_v)
        pltpu.sync_copy(pos_hbm.at[pl.ds(0, POS_ROWS)], pos_v)
        iv = lax.iota(jnp.int32, 16)
        sidx_v[pl.ds(0, 16)] = jnp.where(iv < 3, iv, 0)
        pltpu.async_copy(seg_hbm.at[sidx_v], seg_v, sem_g0).wait()

        def fire_tok(ci, slot):
            t0 = pl.multiple_of(ci * C, C)
            pltpu.async_copy(tok_hbm.at[x_v.at[pl.ds(t0, C)]],
                             acc2.at[pl.ds(slot * C, C)], sem_g[slot])

        def process(ci, slot, fire_next, wait_wb):
            nslot = 1 - slot
            if fire_next:
                if wait_wb:
                    pltpu.make_async_copy(
                        acc2.at[pl.ds(nslot * C, C)], out_hbm.at[pl.ds(0, C)],
                        sem_w[nslot]).wait()
                fire_tok(ci + 1, nslot)
            t0 = pl.multiple_of(ci * C, C)
            tv = t0 + lax.iota(jnp.int32, LANES)
            lv = lax.rem(tv, L)
            xv = x_v[pl.ds(t0, LANES)]
            sv = xseg_v[pl.ds(t0, LANES)]
            pt = jnp.where(xv == 0, 0, lv + 1).astype(jnp.int32)
            for j in range(C):
                rows_m[j] = pt[j]
                rows_m[C + j] = sv[j]
            pltpu.make_async_copy(tok_hbm.at[pl.ds(0, C)],
                                  acc2.at[pl.ds(slot * C, C)],
                                  sem_g[slot]).wait()

            def row_body(tj, carry):
                prow = rows_m[tj]
                srow = rows_m[C + tj]
                for d0 in range(0, H // LANES, 8):
                    sls = [pl.ds((d0 + k) * LANES, LANES) for k in range(8)]
                    vals = [pos_v[prow, sl] + seg_v[srow, sl] for sl in sls]
                    for sl, val in zip(sls, vals):
                        plsc.addupdate(acc2.at[slot * C + tj, sl], val)
                return carry
            lax.fori_loop(0, C, row_body, 0)

            pltpu.async_copy(acc2.at[pl.ds(slot * C, C)],
                             out_hbm.at[pl.ds(base + t0, C)], sem_w[slot])

        fire_tok(0, 0)
        process(0, 0, fire_next=True, wait_wb=False)

        def pair_body(p, carry):
            ci = 1 + 2 * p
            process(ci, 1, fire_next=True, wait_wb=True)
            process(ci + 1, 0, fire_next=True, wait_wb=True)
            return carry
        lax.fori_loop(0, (NCHUNK - 2) // 2, pair_body, 0)

        process(NCHUNK - 1, 1, fire_next=False, wait_wb=False)

        pltpu.make_async_copy(acc2.at[pl.ds(0, C)], out_hbm.at[pl.ds(0, C)],
                              sem_w[0]).wait()
        pltpu.make_async_copy(acc2.at[pl.ds(C, C)], out_hbm.at[pl.ds(0, C)],
                              sem_w[1]).wait()

    return gather_sum


_gather_sum = _build_kernel()


@jax.jit
def kernel(x, x_seg, token_table, pos_table, seg_table):
    out = _gather_sum(x.reshape(-1), x_seg.reshape(-1),
                      token_table, pos_table, seg_table)
    return out.reshape(B, L, H)


# speedup vs baseline: 3.0634x; 1.0894x over previous
import functools

import jax
import jax.numpy as jnp
from jax import lax
from jax.experimental import pallas as pl
from jax.experimental.pallas import tpu as pltpu
from jax.experimental.pallas import tpu_sc as plsc

B, L, H = 1024, 50, 768
NC, NS, LANES = 2, 16, 16
NW = NC * NS
TOK_PER_W = (B * L) // NW
C = 16
NCHUNK = TOK_PER_W // C
NCOMB = 160
PAD_ROW = L * 3
D2 = H // 32


def _build_kernel():
    mesh = plsc.VectorSubcoreMesh(core_axis_name="c", subcore_axis_name="s")

    @functools.partial(
        pl.kernel,
        mesh=mesh,
        out_type=jax.ShapeDtypeStruct((B * L, H), jnp.float32),
        scratch_types=[
            pltpu.VMEM((TOK_PER_W,), jnp.int32),
            pltpu.VMEM((TOK_PER_W,), jnp.int32),
            pltpu.VMEM((NCOMB, H // 2), jnp.int32),
            pltpu.VMEM((16,), jnp.int32),
            pltpu.VMEM((8, H), jnp.float32),
            pltpu.VMEM((2 * C, H), jnp.float32),
            pltpu.SMEM((C,), jnp.int32),
            pltpu.SemaphoreType.DMA,
            pltpu.SemaphoreType.DMA,
            pltpu.SemaphoreType.DMA,
            pltpu.SemaphoreType.DMA,
        ],
    )
    def gather_sum(x_hbm, xseg_hbm, tok_hbm, pos_hbm, seg_hbm, out_hbm,
                   x_v, xseg_v, comb_v, sidx_v, seg_v, acc2, rows_m,
                   sem_g0, sem_g1, sem_w0, sem_w1):
        sem_g = (sem_g0, sem_g1)
        sem_w = (sem_w0, sem_w1)

        wid = lax.axis_index("s") * NC + lax.axis_index("c")
        base = wid * TOK_PER_W
        pltpu.sync_copy(x_hbm.at[pl.ds(base, TOK_PER_W)], x_v)
        pltpu.sync_copy(xseg_hbm.at[pl.ds(base, TOK_PER_W)], xseg_v)

        iv = lax.iota(jnp.int32, 16)
        sidx_v[pl.ds(0, 16)] = jnp.where(iv < 3, iv, 0)
        pltpu.async_copy(seg_hbm.at[sidx_v.at[pl.ds(0, 8)]], seg_v,
                         sem_g0).wait()

        def pack2(a, b):
            ai = lax.bitcast_convert_type(a, jnp.int32) + 0x8000
            bi = lax.bitcast_convert_type(b, jnp.int32) + 0x8000
            hi = jnp.bitwise_and(ai, -65536)
            lo = jnp.bitwise_and(jnp.right_shift(bi, 16), 0xFFFF)
            return jnp.bitwise_or(hi, lo)

        def pad_body(d2, carry):
            co = pl.multiple_of(d2 * 32, 32)
            for s in range(3):
                v = pack2(seg_v[s, pl.ds(co, 16)],
                          seg_v[s, pl.ds(co + 16, 16)])
                comb_v[PAD_ROW + s, pl.ds(d2 * 16, 16)] = v
            return carry
        lax.fori_loop(0, D2, pad_body, 0)

        def build(prow_lo, nrows, stage_base):
            def comb_body(i, carry):
                prow = prow_lo + i
                arow = prow - stage_base
                def col_body(d2, carry2):
                    co = pl.multiple_of(d2 * 32, 32)
                    pa = acc2[arow, pl.ds(co, 16)]
                    pb = acc2[arow, pl.ds(co + 16, 16)]
                    for s in range(3):
                        a = pa + seg_v[s, pl.ds(co, 16)]
                        b = pb + seg_v[s, pl.ds(co + 16, 16)]
                        v = pack2(a, b)
                        comb_v[(prow - 1) * 3 + s, pl.ds(d2 * 16, 16)] = v
                    return carry2
                lax.fori_loop(0, D2, col_body, 0)
                return carry
            lax.fori_loop(0, nrows, comb_body, 0)

        pltpu.sync_copy(pos_hbm.at[pl.ds(0, 32)], acc2)
        build(1, 31, 0)
        pltpu.sync_copy(pos_hbm.at[pl.ds(32, 24)], acc2.at[pl.ds(0, 24)])
        build(32, 19, 32)

        def fire_tok(ci, slot):
            t0 = pl.multiple_of(ci * C, C)
            pltpu.async_copy(tok_hbm.at[x_v.at[pl.ds(t0, C)]],
                             acc2.at[pl.ds(slot * C, C)], sem_g[slot])

        def process(ci, slot, fire_next, wait_wb):
            nslot = 1 - slot
            if fire_next:
                if wait_wb:
                    pltpu.make_async_copy(
                        acc2.at[pl.ds(nslot * C, C)], out_hbm.at[pl.ds(0, C)],
                        sem_w[nslot]).wait()
                fire_tok(ci + 1, nslot)
            t0 = pl.multiple_of(ci * C, C)
            tv = t0 + lax.iota(jnp.int32, LANES)
            lv = lax.rem(tv, L)
            xv = x_v[pl.ds(t0, LANES)]
            sv = xseg_v[pl.ds(t0, LANES)]
            crow = jnp.where(xv == 0, PAD_ROW + sv, lv * 3 + sv)
            crow = crow.astype(jnp.int32)
            for j in range(C):
                rows_m[j] = crow[j]
            pltpu.make_async_copy(tok_hbm.at[pl.ds(0, C)],
                                  acc2.at[pl.ds(slot * C, C)],
                                  sem_g[slot]).wait()

            def row_body(tj, carry):
                cr = rows_m[tj]
                for g0 in range(0, D2, 4):
                    packed = [comb_v[cr, pl.ds((g0 + k) * 16, 16)]
                              for k in range(4)]
                    ab = []
                    for k, v in enumerate(packed):
                        a = lax.bitcast_convert_type(
                            jnp.bitwise_and(v, -65536), jnp.float32)
                        b = lax.bitcast_convert_type(
                            jnp.left_shift(v, 16), jnp.float32)
                        ab.append((a, b))
                    for k, (a, b) in enumerate(ab):
                        co = (g0 + k) * 32
                        plsc.addupdate(
                            acc2.at[slot * C + tj, pl.ds(co, 16)], a)
                        plsc.addupdate(
                            acc2.at[slot * C + tj, pl.ds(co + 16, 16)], b)
                return carry
            lax.fori_loop(0, C, row_body, 0)

            pltpu.async_copy(acc2.at[pl.ds(slot * C, C)],
                             out_hbm.at[pl.ds(base + t0, C)], sem_w[slot])

        fire_tok(0, 0)
        process(0, 0, fire_next=True, wait_wb=False)

        def pair_body(p, carry):
            ci = 1 + 2 * p
            process(ci, 1, fire_next=True, wait_wb=True)
            process(ci + 1, 0, fire_next=True, wait_wb=True)
            return carry
        lax.fori_loop(0, (NCHUNK - 2) // 2, pair_body, 0)

        process(NCHUNK - 1, 1, fire_next=False, wait_wb=False)

        pltpu.make_async_copy(acc2.at[pl.ds(0, C)], out_hbm.at[pl.ds(0, C)],
                              sem_w[0]).wait()
        pltpu.make_async_copy(acc2.at[pl.ds(C, C)], out_hbm.at[pl.ds(0, C)],
                              sem_w[1]).wait()

    return gather_sum


_gather_sum = _build_kernel()


@jax.jit
def kernel(x, x_seg, token_table, pos_table, seg_table):
    out = _gather_sum(x.reshape(-1), x_seg.reshape(-1),
                      token_table, pos_table, seg_table)
    return out.reshape(B, L, H)


# speedup vs baseline: 3.1331x; 1.0228x over previous
import functools

import jax
import jax.numpy as jnp
from jax import lax
from jax.experimental import pallas as pl
from jax.experimental.pallas import tpu as pltpu
from jax.experimental.pallas import tpu_sc as plsc

B, L, H = 1024, 50, 768
NC, NS, LANES = 2, 16, 16
NW = NC * NS
TOK_PER_W = (B * L) // NW
C = 32
NCHUNK = TOK_PER_W // C
NCOMB = 160
PAD_ROW = L * 3
D2 = H // 32


def _build_kernel():
    mesh = plsc.VectorSubcoreMesh(core_axis_name="c", subcore_axis_name="s")

    @functools.partial(
        pl.kernel,
        mesh=mesh,
        out_type=jax.ShapeDtypeStruct((B * L, H), jnp.float32),
        scratch_types=[
            pltpu.VMEM((TOK_PER_W,), jnp.int32),
            pltpu.VMEM((TOK_PER_W,), jnp.int32),
            pltpu.VMEM((NCOMB, H // 2), jnp.int32),
            pltpu.VMEM((16,), jnp.int32),
            pltpu.VMEM((8, H), jnp.float32),
            pltpu.VMEM((2 * C, H), jnp.float32),
            pltpu.SMEM((C,), jnp.int32),
            pltpu.SemaphoreType.DMA,
            pltpu.SemaphoreType.DMA,
            pltpu.SemaphoreType.DMA,
            pltpu.SemaphoreType.DMA,
        ],
    )
    def gather_sum(x_hbm, xseg_hbm, tok_hbm, pos_hbm, seg_hbm, out_hbm,
                   x_v, xseg_v, comb_v, sidx_v, seg_v, acc2, rows_m,
                   sem_g0, sem_g1, sem_w0, sem_w1):
        sem_g = (sem_g0, sem_g1)
        sem_w = (sem_w0, sem_w1)

        wid = lax.axis_index("s") * NC + lax.axis_index("c")
        base = wid * TOK_PER_W
        pltpu.sync_copy(x_hbm.at[pl.ds(base, TOK_PER_W)], x_v)
        pltpu.sync_copy(xseg_hbm.at[pl.ds(base, TOK_PER_W)], xseg_v)

        iv = lax.iota(jnp.int32, 16)
        sidx_v[pl.ds(0, 16)] = jnp.where(iv < 3, iv, 0)
        pltpu.async_copy(seg_hbm.at[sidx_v.at[pl.ds(0, 8)]], seg_v,
                         sem_g0).wait()

        def pack2(a, b):
            ai = lax.bitcast_convert_type(a, jnp.int32) + 0x8000
            bi = lax.bitcast_convert_type(b, jnp.int32) + 0x8000
            hi = jnp.bitwise_and(ai, -65536)
            lo = jnp.bitwise_and(jnp.right_shift(bi, 16), 0xFFFF)
            return jnp.bitwise_or(hi, lo)

        def pad_body(d2, carry):
            co = pl.multiple_of(d2 * 32, 32)
            for s in range(3):
                v = pack2(seg_v[s, pl.ds(co, 16)],
                          seg_v[s, pl.ds(co + 16, 16)])
                comb_v[PAD_ROW + s, pl.ds(d2 * 16, 16)] = v
            return carry
        lax.fori_loop(0, D2, pad_body, 0)

        def build(prow_lo, nrows, stage_base):
            def comb_body(i, carry):
                prow = prow_lo + i
                arow = prow - stage_base
                def col_body(d2, carry2):
                    co = pl.multiple_of(d2 * 32, 32)
                    pa = acc2[arow, pl.ds(co, 16)]
                    pb = acc2[arow, pl.ds(co + 16, 16)]
                    for s in range(3):
                        a = pa + seg_v[s, pl.ds(co, 16)]
                        b = pb + seg_v[s, pl.ds(co + 16, 16)]
                        v = pack2(a, b)
                        comb_v[(prow - 1) * 3 + s, pl.ds(d2 * 16, 16)] = v
                    return carry2
                lax.fori_loop(0, D2, col_body, 0)
                return carry
            lax.fori_loop(0, nrows, comb_body, 0)

        pltpu.sync_copy(pos_hbm.at[pl.ds(0, 32)], acc2.at[pl.ds(0, 32)])
        build(1, 31, 0)
        pltpu.sync_copy(pos_hbm.at[pl.ds(32, 24)], acc2.at[pl.ds(0, 24)])
        build(32, 19, 32)

        def fire_tok(ci, slot):
            t0 = pl.multiple_of(ci * C, C)
            pltpu.async_copy(tok_hbm.at[x_v.at[pl.ds(t0, C)]],
                             acc2.at[pl.ds(slot * C, C)], sem_g[slot])

        def process(ci, slot, fire_next, wait_wb):
            nslot = 1 - slot
            if fire_next:
                if wait_wb:
                    pltpu.make_async_copy(
                        acc2.at[pl.ds(nslot * C, C)], out_hbm.at[pl.ds(0, C)],
                        sem_w[nslot]).wait()
                fire_tok(ci + 1, nslot)
            t0 = pl.multiple_of(ci * C, C)
            for k in range(C // LANES):
                tv = t0 + k * LANES + lax.iota(jnp.int32, LANES)
                lv = lax.rem(tv, L)
                xv = x_v[pl.ds(t0 + k * LANES, LANES)]
                sv = xseg_v[pl.ds(t0 + k * LANES, LANES)]
                crow = jnp.where(xv == 0, PAD_ROW + sv, lv * 3 + sv)
                crow = crow.astype(jnp.int32)
                for j in range(LANES):
                    rows_m[k * LANES + j] = crow[j]
            pltpu.make_async_copy(tok_hbm.at[pl.ds(0, C)],
                                  acc2.at[pl.ds(slot * C, C)],
                                  sem_g[slot]).wait()

            def row_body(tj, carry):
                cr = rows_m[tj]
                for g0 in range(0, D2, 4):
                    packed = [comb_v[cr, pl.ds((g0 + k) * 16, 16)]
                              for k in range(4)]
                    ab = []
                    for k, v in enumerate(packed):
                        a = lax.bitcast_convert_type(
                            jnp.bitwise_and(v, -65536), jnp.float32)
                        b = lax.bitcast_convert_type(
                            jnp.left_shift(v, 16), jnp.float32)
                        ab.append((a, b))
                    for k, (a, b) in enumerate(ab):
                        co = (g0 + k) * 32
                        plsc.addupdate(
                            acc2.at[slot * C + tj, pl.ds(co, 16)], a)
                        plsc.addupdate(
                            acc2.at[slot * C + tj, pl.ds(co + 16, 16)], b)
                return carry
            lax.fori_loop(0, C, row_body, 0)

            pltpu.async_copy(acc2.at[pl.ds(slot * C, C)],
                             out_hbm.at[pl.ds(base + t0, C)], sem_w[slot])

        fire_tok(0, 0)
        process(0, 0, fire_next=True, wait_wb=False)

        def pair_body(p, carry):
            ci = 1 + 2 * p
            process(ci, 1, fire_next=True, wait_wb=True)
            process(ci + 1, 0, fire_next=True, wait_wb=True)
            return carry
        lax.fori_loop(0, (NCHUNK - 2) // 2, pair_body, 0)

        process(NCHUNK - 1, 1, fire_next=False, wait_wb=False)

        pltpu.make_async_copy(acc2.at[pl.ds(0, C)], out_hbm.at[pl.ds(0, C)],
                              sem_w[0]).wait()
        pltpu.make_async_copy(acc2.at[pl.ds(C, C)], out_hbm.at[pl.ds(0, C)],
                              sem_w[1]).wait()

    return gather_sum


_gather_sum = _build_kernel()


@jax.jit
def kernel(x, x_seg, token_table, pos_table, seg_table):
    out = _gather_sum(x.reshape(-1), x_seg.reshape(-1),
                      token_table, pos_table, seg_table)
    return out.reshape(B, L, H)
